# Initial kernel scaffold; baseline (speedup 1.0000x reference)
import jax
import jax.numpy as jnp
from jax.experimental import pallas as pl


def kernel(hidden_states, attention_mask, self_attention_scores, key_layer, tome_size, ln1_g, ln1_b, W1, b1, ln2_g, ln2_b, W2, b2):
    raise NotImplementedError("write your pallas kernel here")


# speedup vs baseline: 2.0925x; 2.0925x over previous
import functools

import jax
import jax.numpy as jnp
from jax.experimental import pallas as pl
from jax.experimental.pallas import tpu as pltpu

_LN_EPS = 1e-5


def _gating_body(x_ref, w1_ref, b1_ref, ln1g_ref, ln1b_ref, ln2g_ref,
                 ln2b_ref, w2_ref, b2_ref, mask_ref, *, tm, seq_len):
    x = x_ref[...]
    mu = jnp.mean(x, axis=-1, keepdims=True)
    xc = x - mu
    var = jnp.mean(xc * xc, axis=-1, keepdims=True)
    h = xc / jnp.sqrt(var + _LN_EPS) * ln1g_ref[...] + ln1b_ref[...]
    h = jax.lax.dot_general(h, w1_ref[...], (((1,), (1,)), ((), ())),
                            preferred_element_type=jnp.float32)
    h = h + b1_ref[...]
    mu2 = jnp.mean(h, axis=-1, keepdims=True)
    hc = h - mu2
    var2 = jnp.mean(hc * hc, axis=-1, keepdims=True)
    h = hc / jnp.sqrt(var2 + _LN_EPS) * ln2g_ref[...] + ln2b_ref[...]
    h = h * 0.5 * (1.0 + jax.lax.erf(h * jnp.float32(0.7071067811865476)))
    s = jax.lax.dot_general(h, w2_ref[...], (((1,), (1,)), ((), ())),
                            preferred_element_type=jnp.float32)
    s = s + b2_ref[...]
    i = pl.program_id(0)
    row = i * tm + jax.lax.broadcasted_iota(jnp.int32, (tm, 1), 0)
    cl = jnp.where(row % seq_len == 0, jnp.float32(100.0), jnp.float32(0.0))
    s0 = s[:, 0:1] + cl
    s1 = s[:, 1:2]
    m = jnp.maximum(s0, s1)
    e0 = jnp.exp(s0 - m)
    e1 = jnp.exp(s1 - m)
    y = e0 / (e0 + e1)
    mask_ref[...] = (y >= 0.5).astype(jnp.float32)


def _gating_mask(hs2d, ln1_g, ln1_b, W1, b1, ln2_g, ln2_b, W2, b2, seq_len):
    n, d = hs2d.shape
    tm = 512
    grid = (n // tm,)
    body = functools.partial(_gating_body, tm=tm, seq_len=seq_len)
    return pl.pallas_call(
        body,
        grid=grid,
        in_specs=[
            pl.BlockSpec((tm, d), lambda i: (i, 0)),
            pl.BlockSpec((d, d), lambda i: (0, 0)),
            pl.BlockSpec((1, d), lambda i: (0, 0)),
            pl.BlockSpec((1, d), lambda i: (0, 0)),
            pl.BlockSpec((1, d), lambda i: (0, 0)),
            pl.BlockSpec((1, d), lambda i: (0, 0)),
            pl.BlockSpec((1, d), lambda i: (0, 0)),
            pl.BlockSpec((2, d), lambda i: (0, 0)),
            pl.BlockSpec((1, 2), lambda i: (0, 0)),
        ],
        out_specs=pl.BlockSpec((tm, 1), lambda i: (i, 0)),
        out_shape=jax.ShapeDtypeStruct((n, 1), jnp.float32),
        compiler_params=pltpu.CompilerParams(
            dimension_semantics=("arbitrary",),
        ),
    )(hs2d, W1, b1.reshape(1, d), ln1_g.reshape(1, d), ln1_b.reshape(1, d),
      ln2_g.reshape(1, d), ln2_b.reshape(1, d), W2, b2.reshape(1, 2))


def kernel(hidden_states, attention_mask, self_attention_scores, key_layer,
           tome_size, ln1_g, ln1_b, W1, b1, ln2_g, ln2_b, W2, b2):
    B, L, D = hidden_states.shape
    hs2d = hidden_states.reshape(B * L, D)
    mask = _gating_mask(hs2d, ln1_g, ln1_b, W1, b1, ln2_g, ln2_b, W2, b2, L)
    learnable_01mask = mask.reshape(B, L)
    tome_size_new = jnp.ones((B, L, 1), dtype=hidden_states.dtype)
    return (hidden_states, attention_mask, tome_size_new, learnable_01mask)
